# Initial kernel scaffold; baseline (speedup 1.0000x reference)
import jax
import jax.numpy as jnp
from jax.experimental import pallas as pl


def kernel(x, embedding_list):
    raise NotImplementedError("write your pallas kernel here")


# speedup vs baseline: 2.3982x; 2.3982x over previous
import functools

import jax
import jax.numpy as jnp
from jax import lax
from jax.experimental import pallas as pl
from jax.experimental.pallas import tpu as pltpu
from jax.experimental.pallas import tpu_sc as plsc

_N_ATOMS = 819200
_N_FEATURES = 64
_NUM_WORKERS = 32
_B_PER_W = _N_ATOMS // _NUM_WORKERS
_CHUNK = 800
_N_CHUNKS = _B_PER_W // _CHUNK


def _emb_body(x_hbm, table_hbm, out_hbm, idx_v, rows_v, sem):
    wid = lax.axis_index("s") * 2 + lax.axis_index("c")
    base = wid * _B_PER_W

    def chunk(i, carry):
        off = base + i * _CHUNK
        pltpu.sync_copy(x_hbm.at[pl.ds(off, _CHUNK)], idx_v)
        pltpu.async_copy(table_hbm.at[idx_v], rows_v, sem).wait()
        pltpu.sync_copy(rows_v, out_hbm.at[pl.ds(off, _CHUNK)])
        return carry

    lax.fori_loop(0, _N_CHUNKS, chunk, 0)


@jax.jit
def kernel(x, embedding_list):
    run = pl.kernel(
        _emb_body,
        out_type=jax.ShapeDtypeStruct((_N_ATOMS, _N_FEATURES), jnp.float32),
        mesh=plsc.VectorSubcoreMesh(core_axis_name="c", subcore_axis_name="s"),
        scratch_types=[
            pltpu.VMEM((_CHUNK,), jnp.int32),
            pltpu.VMEM((_CHUNK, _N_FEATURES), jnp.float32),
            pltpu.SemaphoreType.DMA,
        ],
        compiler_params=pltpu.CompilerParams(use_tc_tiling_on_sc=False),
    )
    return run(x, embedding_list)


# speedup vs baseline: 5.5414x; 2.3106x over previous
import jax
import jax.numpy as jnp
from jax import lax
from jax.experimental import pallas as pl
from jax.experimental.pallas import tpu as pltpu
from jax.experimental.pallas import tpu_sc as plsc

_N_ATOMS = 819200
_N_FEATURES = 64
_TABLE_ROWS = 83
_NC = 2
_NS = 16
_NUM_WORKERS = _NC * _NS
_B_PER_W = _N_ATOMS // _NUM_WORKERS
_CHUNK = 800
_N_CHUNKS = _B_PER_W // _CHUNK


def _emb_body(x_hbm, table_hbm, out_hbm, table_sh, idx_v, rows0, rows1,
              semb0, semb1, semc0, semc1):
    cid = lax.axis_index("c")
    sid = lax.axis_index("s")
    wid = sid * _NC + cid
    base = wid * _B_PER_W

    @pl.when(sid == 0)
    def _():
        pltpu.sync_copy(table_hbm, table_sh)

    pltpu.sync_copy(x_hbm.at[pl.ds(base, _B_PER_W)], idx_v)
    plsc.subcore_barrier()

    rows = (rows0, rows1)
    semb = (semb0, semb1)
    semc = (semc0, semc1)

    def gather(i, b):
        return pltpu.async_copy(
            table_sh.at[idx_v.at[pl.ds(i * _CHUNK, _CHUNK)]], rows[b], semb[b])

    def put(i, b):
        return pltpu.async_copy(
            rows[b], out_hbm.at[pl.ds(base + i * _CHUNK, _CHUNK)], semc[b])

    g0 = gather(0, 0)
    g1 = gather(1, 1)
    g0.wait()
    put(0, 0)

    def pair(j, carry):
        i0 = 2 * j
        for b in range(2):
            i = i0 + b
            pltpu.make_async_copy(rows[b], out_hbm.at[pl.ds(0, _CHUNK)],
                                  semc[b]).wait()
            gather(i, b)
            pltpu.make_async_copy(
                table_sh.at[idx_v.at[pl.ds(0, _CHUNK)]], rows[1 - b],
                semb[1 - b]).wait()
            put(i - 1, 1 - b)
        return carry

    lax.fori_loop(1, _N_CHUNKS // 2, pair, 0)

    pltpu.make_async_copy(table_sh.at[idx_v.at[pl.ds(0, _CHUNK)]], rows1,
                          semb1).wait()
    put(_N_CHUNKS - 1, 1)
    pltpu.make_async_copy(rows0, out_hbm.at[pl.ds(0, _CHUNK)], semc0).wait()
    pltpu.make_async_copy(rows1, out_hbm.at[pl.ds(0, _CHUNK)], semc1).wait()


@jax.jit
def kernel(x, embedding_list):
    run = pl.kernel(
        _emb_body,
        out_type=jax.ShapeDtypeStruct((_N_ATOMS, _N_FEATURES), jnp.float32),
        mesh=plsc.VectorSubcoreMesh(core_axis_name="c", subcore_axis_name="s"),
        scratch_types=[
            pltpu.VMEM_SHARED((_TABLE_ROWS, _N_FEATURES), jnp.float32),
            pltpu.VMEM((_B_PER_W,), jnp.int32),
            pltpu.VMEM((_CHUNK, _N_FEATURES), jnp.float32),
            pltpu.VMEM((_CHUNK, _N_FEATURES), jnp.float32),
            pltpu.SemaphoreType.DMA,
            pltpu.SemaphoreType.DMA,
            pltpu.SemaphoreType.DMA,
            pltpu.SemaphoreType.DMA,
        ],
        compiler_params=pltpu.CompilerParams(use_tc_tiling_on_sc=False),
    )
    return run(x, embedding_list)
